# baseline (speedup 1.0000x reference)
import jax
import jax.numpy as jnp
from jax.experimental import pallas as pl
from jax.experimental.pallas import tpu as pltpu


def _qsample_body(t_ref, a_tbl, s_tbl, x0_ref, n_ref, xt_ref):
    i = pl.program_id(0)
    tt = t_ref[0, i]
    a = a_tbl[0, tt]
    s = s_tbl[0, tt]
    xt_ref[...] = a * x0_ref[...] + s * n_ref[...]


def kernel(x0, t, noise, sqrt_alphas_bar, sqrt_one_minus_alphas_bar):
    b = x0.shape[0]
    row = x0.size // b
    sub = row // 128
    x0f = x0.reshape(b, sub, 128)
    nf = noise.reshape(b, sub, 128)
    xt = pl.pallas_call(
        _qsample_body,
        grid=(b,),
        in_specs=[
            pl.BlockSpec(memory_space=pltpu.SMEM),
            pl.BlockSpec(memory_space=pltpu.SMEM),
            pl.BlockSpec(memory_space=pltpu.SMEM),
            pl.BlockSpec((1, sub, 128), lambda i: (i, 0, 0)),
            pl.BlockSpec((1, sub, 128), lambda i: (i, 0, 0)),
        ],
        out_specs=pl.BlockSpec((1, sub, 128), lambda i: (i, 0, 0)),
        out_shape=jax.ShapeDtypeStruct((b, sub, 128), jnp.float32),
    )(
        t.reshape(1, b).astype(jnp.int32),
        sqrt_alphas_bar.reshape(1, -1),
        sqrt_one_minus_alphas_bar.reshape(1, -1),
        x0f,
        nf,
    )
    return xt.reshape(x0.shape), noise
